# Initial kernel scaffold; baseline (speedup 1.0000x reference)
import jax
import jax.numpy as jnp
from jax.experimental import pallas as pl


def kernel(center_word, context_word, negative_samples, embed_v, embed_u):
    raise NotImplementedError("write your pallas kernel here")


# speedup vs baseline: 1.7091x; 1.7091x over previous
import jax
import jax.numpy as jnp
from jax import lax
from jax.experimental import pallas as pl
from jax.experimental.pallas import tpu as pltpu
from jax.experimental.pallas import tpu_sc as plsc

VOCAB = 1000000
EMBED = 64
BATCH = 16384
NUM_NEG = 5
NSCORE = 1 + NUM_NEG
LANES = 16

NC = 2
NS = 16
NW = NC * NS
BPW = BATCH // NW
CHUNK = 128
NCHUNK = BPW // CHUNK
PW = NSCORE * LANES


def _sc_scores_body(cidx_hbm, xidx_hbm, nidx_hbm, ev_hbm, eu_hbm, out_hbm,
                    cidx, xidx, nidx, vrows, urows, nrows, scores, sem):
    c = lax.axis_index("c")
    s = lax.axis_index("s")
    wid = s * NC + c
    base = wid * BPW

    pltpu.sync_copy(cidx_hbm.at[pl.ds(base, BPW)], cidx)
    pltpu.sync_copy(xidx_hbm.at[pl.ds(base, BPW)], xidx)
    pltpu.sync_copy(nidx_hbm.at[pl.ds(base * NUM_NEG, BPW * NUM_NEG)], nidx)

    for j in range(NCHUNK):
        handles = [
            pltpu.async_copy(ev_hbm.at[cidx.at[pl.ds(j * CHUNK, CHUNK)]],
                             vrows, sem),
            pltpu.async_copy(eu_hbm.at[xidx.at[pl.ds(j * CHUNK, CHUNK)]],
                             urows, sem),
        ]
        for t in range(NUM_NEG):
            handles.append(pltpu.async_copy(
                eu_hbm.at[nidx.at[pl.ds((j * NUM_NEG + t) * CHUNK, CHUNK)]],
                nrows.at[pl.ds(t * CHUNK, CHUNK)], sem))
        for h in handles:
            h.wait()

        def elem(e, carry, j=j):
            u = [urows[e, pl.ds(16 * t, 16)] for t in range(4)]
            off = (j * CHUNK + e) * PW
            accp = vrows[e, pl.ds(0, 16)] * u[0]
            for t in range(1, 4):
                accp += vrows[e, pl.ds(16 * t, 16)] * u[t]
            scores[pl.ds(off, 16)] = accp
            for k in range(NUM_NEG):
                r = e * NUM_NEG + k
                accn = nrows[r, pl.ds(0, 16)] * u[0]
                for t in range(1, 4):
                    accn += nrows[r, pl.ds(16 * t, 16)] * u[t]
                scores[pl.ds(off + (1 + k) * 16, 16)] = -accn
            return carry

        lax.fori_loop(0, CHUNK, elem, 0)

    pltpu.sync_copy(scores, out_hbm.at[pl.ds(base * PW, BPW * PW)])


def _sc_scores(center, context, negflat, embed_v, embed_u):
    mesh = plsc.VectorSubcoreMesh(core_axis_name="c", subcore_axis_name="s")
    f = pl.kernel(
        _sc_scores_body,
        out_type=jax.ShapeDtypeStruct((BATCH * PW,), jnp.float32),
        mesh=mesh,
        compiler_params=pltpu.CompilerParams(use_tc_tiling_on_sc=False),
        scratch_types=[
            pltpu.VMEM((BPW,), jnp.int32),
            pltpu.VMEM((BPW,), jnp.int32),
            pltpu.VMEM((BPW * NUM_NEG,), jnp.int32),
            pltpu.VMEM((CHUNK, EMBED), jnp.float32),
            pltpu.VMEM((CHUNK, EMBED), jnp.float32),
            pltpu.VMEM((CHUNK * NUM_NEG, EMBED), jnp.float32),
            pltpu.VMEM((BPW * PW,), jnp.float32),
            pltpu.SemaphoreType.DMA,
        ],
    )
    return f(center, context, negflat, embed_v, embed_u)


def _loss_body(p_ref, o_ref):
    x = p_ref[...]
    g = lax.broadcasted_iota(jnp.int32, (128, 8), 0) // 16
    t = lax.broadcasted_iota(jnp.int32, (128, 8), 1)
    m = jnp.where(g == t, 1.0, 0.0).astype(jnp.float32)
    s = jax.lax.dot_general(x, m, (((1,), (0,)), ((), ())),
                            preferred_element_type=jnp.float32)
    o_ref[0, 0] = jnp.sum(jnp.log(jax.nn.sigmoid(s))) * (-1.0 / BATCH)


def kernel(center_word, context_word, negative_samples, embed_v, embed_u):
    center = center_word.astype(jnp.int32)
    context = context_word.astype(jnp.int32)
    negflat = negative_samples.astype(jnp.int32).reshape(-1)
    partials = _sc_scores(center, context, negflat, embed_v, embed_u)
    partials2d = partials.reshape(BATCH * PW // 128, 128)
    loss = pl.pallas_call(
        _loss_body,
        out_shape=jax.ShapeDtypeStruct((1, 1), jnp.float32),
        out_specs=pl.BlockSpec(memory_space=pltpu.SMEM),
    )(partials2d)
    return loss[0, 0]


# speedup vs baseline: 2.5511x; 1.4927x over previous
import jax
import jax.numpy as jnp
from jax import lax
from jax.experimental import pallas as pl
from jax.experimental.pallas import tpu as pltpu
from jax.experimental.pallas import tpu_sc as plsc

VOCAB = 1000000
EMBED = 64
BATCH = 16384
NUM_NEG = 5
NSCORE = 1 + NUM_NEG
LANES = 16

NC = 2
NS = 16
NW = NC * NS
BPW = BATCH // NW
CHUNK = 64
NCHUNK = BPW // CHUNK
PW = NSCORE * LANES


def _sc_scores_body(cidx_hbm, xidx_hbm, nidx_hbm, ev_hbm, eu_hbm, out_hbm,
                    cidx, xidx, nidx, vrows, urows, nrows, scores, sem):
    c = lax.axis_index("c")
    s = lax.axis_index("s")
    wid = s * NC + c
    base = wid * BPW

    pltpu.sync_copy(cidx_hbm.at[pl.ds(base, BPW)], cidx)
    pltpu.sync_copy(xidx_hbm.at[pl.ds(base, BPW)], xidx)
    pltpu.sync_copy(nidx_hbm.at[pl.ds(base * NUM_NEG, BPW * NUM_NEG)], nidx)

    for j in range(NCHUNK):
        def issue(g, carry, j=j):
            cv = cidx[pl.ds(j * CHUNK + g * 16, 16)]
            xv = xidx[pl.ds(j * CHUNK + g * 16, 16)]
            nv = [nidx[pl.ds((j * CHUNK) * NUM_NEG + g * 80 + 16 * i, 16)]
                  for i in range(NUM_NEG)]
            for l in range(16):
                e = g * 16 + l
                pltpu.async_copy(ev_hbm.at[cv[l]], vrows.at[e], sem)
                pltpu.async_copy(eu_hbm.at[xv[l]], urows.at[e], sem)
                for k in range(NUM_NEG):
                    p = l * NUM_NEG + k
                    pltpu.async_copy(eu_hbm.at[nv[p // 16][p % 16]],
                                     nrows.at[e * NUM_NEG + k], sem)
            return carry

        lax.fori_loop(0, CHUNK // 16, issue, 0)

        def drain(e, carry):
            for _ in range(1 + 1 + NUM_NEG):
                pltpu.make_async_copy(ev_hbm.at[0], vrows.at[0], sem).wait()
            return carry

        lax.fori_loop(0, CHUNK, drain, 0)

        def elem(e, carry, j=j):
            u = [urows[e, pl.ds(16 * t, 16)] for t in range(4)]
            off = (j * CHUNK + e) * PW
            accp = vrows[e, pl.ds(0, 16)] * u[0]
            for t in range(1, 4):
                accp += vrows[e, pl.ds(16 * t, 16)] * u[t]
            scores[pl.ds(off, 16)] = accp
            for k in range(NUM_NEG):
                r = e * NUM_NEG + k
                accn = nrows[r, pl.ds(0, 16)] * u[0]
                for t in range(1, 4):
                    accn += nrows[r, pl.ds(16 * t, 16)] * u[t]
                scores[pl.ds(off + (1 + k) * 16, 16)] = -accn
            return carry

        lax.fori_loop(0, CHUNK, elem, 0)

    pltpu.sync_copy(scores, out_hbm.at[pl.ds(base * PW, BPW * PW)])


def _sc_scores(center, context, negflat, embed_v, embed_u):
    mesh = plsc.VectorSubcoreMesh(core_axis_name="c", subcore_axis_name="s")
    f = pl.kernel(
        _sc_scores_body,
        out_type=jax.ShapeDtypeStruct((BATCH * PW,), jnp.float32),
        mesh=mesh,
        scratch_types=[
            pltpu.VMEM((BPW,), jnp.int32),
            pltpu.VMEM((BPW,), jnp.int32),
            pltpu.VMEM((BPW * NUM_NEG,), jnp.int32),
            pltpu.VMEM((CHUNK, EMBED), jnp.float32),
            pltpu.VMEM((CHUNK, EMBED), jnp.float32),
            pltpu.VMEM((CHUNK * NUM_NEG, EMBED), jnp.float32),
            pltpu.VMEM((BPW * PW,), jnp.float32),
            pltpu.SemaphoreType.DMA,
        ],
    )
    return f(center, context, negflat, embed_v, embed_u)


def _loss_body(p_ref, o_ref):
    x = p_ref[...]
    g = lax.broadcasted_iota(jnp.int32, (128, 8), 0) // 16
    t = lax.broadcasted_iota(jnp.int32, (128, 8), 1)
    m = jnp.where(g == t, 1.0, 0.0).astype(jnp.float32)
    s = jax.lax.dot_general(x, m, (((1,), (0,)), ((), ())),
                            preferred_element_type=jnp.float32)
    o_ref[0, 0] = jnp.sum(jnp.log(jax.nn.sigmoid(s))) * (-1.0 / BATCH)


def kernel(center_word, context_word, negative_samples, embed_v, embed_u):
    center = center_word.astype(jnp.int32)
    context = context_word.astype(jnp.int32)
    negflat = negative_samples.astype(jnp.int32).reshape(-1)
    partials = _sc_scores(center, context, negflat, embed_v, embed_u)
    partials2d = partials.reshape(BATCH * PW // 128, 128)
    loss = pl.pallas_call(
        _loss_body,
        out_shape=jax.ShapeDtypeStruct((1, 1), jnp.float32),
        out_specs=pl.BlockSpec(memory_space=pltpu.SMEM),
    )(partials2d)
    return loss[0, 0]
